# baseline (speedup 1.0000x reference)
import functools

import jax
import jax.numpy as jnp
from jax.experimental import pallas as pl
from jax.experimental.pallas import tpu as pltpu

Q, K, D, DC, B, T = 8, 1024, 256, 32, 8, 2048
TILE_T = 1024
NCHAIN = 1
HALF = TILE_T // NCHAIN
COMMIT, CBW = 0.25, 1.0


def _cbn_aug_kernel(cb_ref, aug_ref):
    for i in range(Q):
        cb_i = cb_ref[i]
        cn = jnp.sqrt(jnp.sum(cb_i * cb_i, axis=1, keepdims=True))
        cbn = cb_i / jnp.maximum(cn, 1e-12)
        cbsq = jnp.sum(cbn * cbn, axis=1, keepdims=True)
        aug_ref[i] = jnp.concatenate(
            [cbn, jnp.broadcast_to(-(0.5 / 8) * cbsq, (K, 8))], axis=1)


def _rvq_kernel(x_ref, win_ref, bin_ref, wout_ref, bout_ref, aug_ref, cbt_ref,
                qout_ref, allq_ref, idxo_ref, losso_ref, sc0_ref, sc1_ref):
    b = pl.program_id(0)
    tt = pl.program_id(1)
    scs = (sc0_ref, sc1_ref)
    for h in range(NCHAIN):
        scs[h][DC:, :] = jnp.ones((8, HALF), jnp.float32)

    resid = [x_ref[0, :, h * HALF:(h + 1) * HALF] for h in range(NCHAIN)]
    qacc = [jnp.zeros((D, HALF), jnp.float32) for _ in range(NCHAIN)]
    partials = [[] for _ in range(NCHAIN)]
    for i in range(Q):
        for h in range(NCHAIN):
            cols = pl.ds(h * HALF, HALF)
            ze = jnp.dot(win_ref[i], resid[h],
                         preferred_element_type=jnp.float32) + bin_ref[i][:, None]
            n = jnp.sqrt(jnp.sum(ze * ze, axis=0, keepdims=True))
            scs[h][:DC, :] = ze / jnp.maximum(n, 1e-12)
            negdist = jnp.dot(aug_ref[i], scs[h][...],
                              preferred_element_type=jnp.float32)
            idx = jnp.argmax(negdist, axis=0)
            idxo_ref[0, i, cols] = idx
            onehot = (jax.lax.broadcasted_iota(jnp.int32, (K, HALF), 0)
                      == idx[None, :]).astype(jnp.float32)
            zq = jnp.dot(cbt_ref[i], onehot,
                         preferred_element_type=jnp.float32)
            diff = ze - zq
            partials[h].append(jnp.sum(diff * diff))
            out = jnp.dot(wout_ref[i], zq,
                          preferred_element_type=jnp.float32) + bout_ref[i][:, None]
            allq_ref[i, 0, :, cols] = out
            qacc[h] = qacc[h] + out
            resid[h] = resid[h] - out
    for h in range(NCHAIN):
        qout_ref[0, :, pl.ds(h * HALF, HALF)] = qacc[h]

    loss_tile = (jnp.stack([sum(ps) for ps in zip(*partials)])[:, None]
                 * jnp.ones((1, 128), jnp.float32))

    @pl.when(jnp.logical_and(b == 0, tt == 0))
    def _init():
        losso_ref[...] = jnp.zeros_like(losso_ref)

    losso_ref[...] += loss_tile


@functools.partial(jax.jit, static_argnames=())
def kernel(x, Win, bin_, Wout, bout, codebook):
    cbT = jnp.transpose(codebook, (0, 2, 1))
    aug = pl.pallas_call(
        _cbn_aug_kernel,
        out_shape=jax.ShapeDtypeStruct((Q, K, DC + 8), jnp.float32),
    )(codebook)
    nt = T // TILE_T
    grid = (B, nt)
    qout, allq, idxo, lossq = pl.pallas_call(
        _rvq_kernel,
        grid=grid,
        in_specs=[
            pl.BlockSpec((1, D, TILE_T), lambda b, t: (b, 0, t)),
            pl.BlockSpec((Q, DC, D), lambda b, t: (0, 0, 0)),
            pl.BlockSpec((Q, DC), lambda b, t: (0, 0)),
            pl.BlockSpec((Q, D, DC), lambda b, t: (0, 0, 0)),
            pl.BlockSpec((Q, D), lambda b, t: (0, 0)),
            pl.BlockSpec((Q, K, DC + 8), lambda b, t: (0, 0, 0)),
            pl.BlockSpec((Q, DC, K), lambda b, t: (0, 0, 0)),
        ],
        out_specs=[
            pl.BlockSpec((1, D, TILE_T), lambda b, t: (b, 0, t)),
            pl.BlockSpec((Q, 1, D, TILE_T), lambda b, t: (0, b, 0, t)),
            pl.BlockSpec((1, Q, TILE_T), lambda b, t: (b, 0, t)),
            pl.BlockSpec((Q, 128), lambda b, t: (0, 0)),
        ],
        out_shape=[
            jax.ShapeDtypeStruct((B, D, T), jnp.float32),
            jax.ShapeDtypeStruct((Q, B, D, T), jnp.float32),
            jax.ShapeDtypeStruct((B, Q, T), jnp.int32),
            jax.ShapeDtypeStruct((Q, 128), jnp.float32),
        ],
        scratch_shapes=[
            pltpu.VMEM((DC + 8, HALF), jnp.float32),
            pltpu.VMEM((DC + 8, HALF), jnp.float32),
        ],
    )(x, Win, bin_, Wout, bout, aug, cbT)
    losses = lossq[:, 0] * ((COMMIT + CBW) / (B * DC * T))
    indices = jnp.transpose(idxo, (1, 0, 2))
    return qout, indices, losses, allq


# speedup vs baseline: 1.0585x; 1.0585x over previous
import functools

import jax
import jax.numpy as jnp
from jax.experimental import pallas as pl
from jax.experimental.pallas import tpu as pltpu

Q, K, D, DC, B, T = 8, 1024, 256, 32, 8, 2048
TILE_T = 2048
NCHAIN = 1
HALF = TILE_T // NCHAIN
COMMIT, CBW = 0.25, 1.0


def _cbn_aug_kernel(cb_ref, aug_ref):
    for i in range(Q):
        cb_i = cb_ref[i]
        cn = jnp.sqrt(jnp.sum(cb_i * cb_i, axis=1, keepdims=True))
        cbn = cb_i / jnp.maximum(cn, 1e-12)
        cbsq = jnp.sum(cbn * cbn, axis=1, keepdims=True)
        aug_ref[i] = jnp.concatenate(
            [cbn, jnp.broadcast_to(-(0.5 / 8) * cbsq, (K, 8))], axis=1)


def _rvq_kernel(x_ref, win_ref, bin_ref, wout_ref, bout_ref, aug_ref, cbt_ref,
                qout_ref, allq_ref, idxo_ref, losso_ref, sc0_ref, sc1_ref):
    b = pl.program_id(0)
    tt = pl.program_id(1)
    scs = (sc0_ref, sc1_ref)
    for h in range(NCHAIN):
        scs[h][DC:, :] = jnp.ones((8, HALF), jnp.float32)

    resid = [x_ref[0, :, h * HALF:(h + 1) * HALF] for h in range(NCHAIN)]
    qacc = [jnp.zeros((D, HALF), jnp.float32) for _ in range(NCHAIN)]
    partials = [[] for _ in range(NCHAIN)]
    for i in range(Q):
        for h in range(NCHAIN):
            cols = pl.ds(h * HALF, HALF)
            ze = jnp.dot(win_ref[i], resid[h],
                         preferred_element_type=jnp.float32) + bin_ref[i][:, None]
            n = jnp.sqrt(jnp.sum(ze * ze, axis=0, keepdims=True))
            scs[h][:DC, :] = ze / jnp.maximum(n, 1e-12)
            negdist = jnp.dot(aug_ref[i], scs[h][...],
                              preferred_element_type=jnp.float32)
            idx = jnp.argmax(negdist, axis=0)
            idxo_ref[0, i, cols] = idx
            onehot = (jax.lax.broadcasted_iota(jnp.int32, (K, HALF), 0)
                      == idx[None, :]).astype(jnp.float32)
            zq = jnp.dot(cbt_ref[i], onehot,
                         preferred_element_type=jnp.float32)
            diff = ze - zq
            partials[h].append(jnp.sum(diff * diff))
            out = jnp.dot(wout_ref[i], zq,
                          preferred_element_type=jnp.float32) + bout_ref[i][:, None]
            allq_ref[i, 0, :, cols] = out
            qacc[h] = qacc[h] + out
            resid[h] = resid[h] - out
    for h in range(NCHAIN):
        qout_ref[0, :, pl.ds(h * HALF, HALF)] = qacc[h]

    loss_tile = (jnp.stack([sum(ps) for ps in zip(*partials)])[:, None]
                 * jnp.ones((1, 128), jnp.float32))

    @pl.when(jnp.logical_and(b == 0, tt == 0))
    def _init():
        losso_ref[...] = jnp.zeros_like(losso_ref)

    losso_ref[...] += loss_tile


@functools.partial(jax.jit, static_argnames=())
def kernel(x, Win, bin_, Wout, bout, codebook):
    cbT = jnp.transpose(codebook, (0, 2, 1))
    aug = pl.pallas_call(
        _cbn_aug_kernel,
        out_shape=jax.ShapeDtypeStruct((Q, K, DC + 8), jnp.float32),
    )(codebook)
    nt = T // TILE_T
    grid = (B, nt)
    qout, allq, idxo, lossq = pl.pallas_call(
        _rvq_kernel,
        grid=grid,
        in_specs=[
            pl.BlockSpec((1, D, TILE_T), lambda b, t: (b, 0, t)),
            pl.BlockSpec((Q, DC, D), lambda b, t: (0, 0, 0)),
            pl.BlockSpec((Q, DC), lambda b, t: (0, 0)),
            pl.BlockSpec((Q, D, DC), lambda b, t: (0, 0, 0)),
            pl.BlockSpec((Q, D), lambda b, t: (0, 0)),
            pl.BlockSpec((Q, K, DC + 8), lambda b, t: (0, 0, 0)),
            pl.BlockSpec((Q, DC, K), lambda b, t: (0, 0, 0)),
        ],
        out_specs=[
            pl.BlockSpec((1, D, TILE_T), lambda b, t: (b, 0, t)),
            pl.BlockSpec((Q, 1, D, TILE_T), lambda b, t: (0, b, 0, t)),
            pl.BlockSpec((1, Q, TILE_T), lambda b, t: (b, 0, t)),
            pl.BlockSpec((Q, 128), lambda b, t: (0, 0)),
        ],
        out_shape=[
            jax.ShapeDtypeStruct((B, D, T), jnp.float32),
            jax.ShapeDtypeStruct((Q, B, D, T), jnp.float32),
            jax.ShapeDtypeStruct((B, Q, T), jnp.int32),
            jax.ShapeDtypeStruct((Q, 128), jnp.float32),
        ],
        scratch_shapes=[
            pltpu.VMEM((DC + 8, HALF), jnp.float32),
            pltpu.VMEM((DC + 8, HALF), jnp.float32),
        ],
    )(x, Win, bin_, Wout, bout, aug, cbT)
    losses = lossq[:, 0] * ((COMMIT + CBW) / (B * DC * T))
    indices = jnp.transpose(idxo, (1, 0, 2))
    return qout, indices, losses, allq


# speedup vs baseline: 1.0849x; 1.0250x over previous
import functools

import jax
import jax.numpy as jnp
from jax.experimental import pallas as pl
from jax.experimental.pallas import tpu as pltpu

Q, K, D, DC, B, T = 8, 1024, 256, 32, 8, 2048
TILE_T = 2048
COMMIT, CBW = 0.25, 1.0


def _cbn_aug_kernel(cb_ref, aug_ref):
    for i in range(Q):
        cb_i = cb_ref[i]
        cn = jnp.sqrt(jnp.sum(cb_i * cb_i, axis=1, keepdims=True))
        cbn = cb_i / jnp.maximum(cn, 1e-12)
        cbsq = jnp.sum(cbn * cbn, axis=1, keepdims=True)
        aug_ref[i] = jnp.concatenate(
            [cbn, jnp.broadcast_to(-(0.5 / 8) * cbsq, (K, 8))], axis=1)


def _rvq_kernel(x_ref, win_ref, bin_ref, wout_ref, bout_ref, aug_ref, cbt_ref,
                qout_ref, allq_ref, idxo_ref, losso_ref, sc_ref):
    b = pl.program_id(0)
    tt = pl.program_id(1)
    sc_ref[DC:, :] = jnp.ones((8, TILE_T), jnp.float32)

    resid = x_ref[0]
    partials = []
    for i in range(Q):
        ze = jnp.dot(win_ref[i], resid,
                     preferred_element_type=jnp.float32) + bin_ref[i][:, None]
        n = jnp.sqrt(jnp.sum(ze * ze, axis=0, keepdims=True))
        sc_ref[:DC, :] = ze / jnp.maximum(n, 1e-12)
        negdist = jnp.dot(aug_ref[i], sc_ref[...],
                          preferred_element_type=jnp.float32)
        idx = jnp.argmax(negdist, axis=0)
        idxo_ref[0, i, :] = idx
        onehot = (jax.lax.broadcasted_iota(jnp.int32, (K, TILE_T), 0)
                  == idx[None, :]).astype(jnp.float32)
        zq = jnp.dot(cbt_ref[i], onehot,
                     preferred_element_type=jnp.float32)
        diff = ze - zq
        partials.append(jnp.sum(diff * diff))
        out = jnp.dot(wout_ref[i], zq,
                      preferred_element_type=jnp.float32) + bout_ref[i][:, None]
        allq_ref[i, 0] = out
        resid = resid - out
    qout_ref[0] = x_ref[0] - resid

    loss_tile = jnp.stack(partials)[:, None] * jnp.ones((1, 128), jnp.float32)

    @pl.when(jnp.logical_and(b == 0, tt == 0))
    def _init():
        losso_ref[...] = jnp.zeros_like(losso_ref)

    losso_ref[...] += loss_tile


@functools.partial(jax.jit, static_argnames=())
def kernel(x, Win, bin_, Wout, bout, codebook):
    cbT = jnp.transpose(codebook, (0, 2, 1))
    aug = pl.pallas_call(
        _cbn_aug_kernel,
        out_shape=jax.ShapeDtypeStruct((Q, K, DC + 8), jnp.float32),
    )(codebook)
    nt = T // TILE_T
    grid = (B, nt)
    qout, allq, idxo, lossq = pl.pallas_call(
        _rvq_kernel,
        grid=grid,
        in_specs=[
            pl.BlockSpec((1, D, TILE_T), lambda b, t: (b, 0, t)),
            pl.BlockSpec((Q, DC, D), lambda b, t: (0, 0, 0)),
            pl.BlockSpec((Q, DC), lambda b, t: (0, 0)),
            pl.BlockSpec((Q, D, DC), lambda b, t: (0, 0, 0)),
            pl.BlockSpec((Q, D), lambda b, t: (0, 0)),
            pl.BlockSpec((Q, K, DC + 8), lambda b, t: (0, 0, 0)),
            pl.BlockSpec((Q, DC, K), lambda b, t: (0, 0, 0)),
        ],
        out_specs=[
            pl.BlockSpec((1, D, TILE_T), lambda b, t: (b, 0, t)),
            pl.BlockSpec((Q, 1, D, TILE_T), lambda b, t: (0, b, 0, t)),
            pl.BlockSpec((1, Q, TILE_T), lambda b, t: (b, 0, t)),
            pl.BlockSpec((Q, 128), lambda b, t: (0, 0)),
        ],
        out_shape=[
            jax.ShapeDtypeStruct((B, D, T), jnp.float32),
            jax.ShapeDtypeStruct((Q, B, D, T), jnp.float32),
            jax.ShapeDtypeStruct((B, Q, T), jnp.int32),
            jax.ShapeDtypeStruct((Q, 128), jnp.float32),
        ],
        scratch_shapes=[
            pltpu.VMEM((DC + 8, TILE_T), jnp.float32),
        ],
    )(x, Win, bin_, Wout, bout, aug, cbT)
    losses = lossq[:, 0] * ((COMMIT + CBW) / (B * DC * T))
    indices = jnp.transpose(idxo, (1, 0, 2))
    return qout, indices, losses, allq
